# Initial kernel scaffold; baseline (speedup 1.0000x reference)
import jax
import jax.numpy as jnp
from jax.experimental import pallas as pl


def kernel(x, W1, g1, b1, W2, g2, b2, W3, g3, b3, W4, g4, b4, W5, g5, b5, L1, g6, b6, L2, L2b, g7, b7, L3, L3b):
    raise NotImplementedError("write your pallas kernel here")


# speedup vs baseline: 5.1071x; 5.1071x over previous
import jax
import jax.numpy as jnp
from jax.experimental import pallas as pl
from jax.experimental.pallas import tpu as pltpu

N = 1024
K = 20
NEG = -3.0e38
HI = jax.lax.Precision.HIGHEST


def _lrelu(x):
    return jnp.where(x >= 0, x, 0.2 * x)


def _stage_body(x_ref, w_ref, s_ref, b_ref, out_ref, score_ref, m_ref):
    xt = x_ref[0]
    o = w_ref.shape[1]
    xb = xt.astype(jnp.bfloat16)

    g = jax.lax.dot_general(xb, xb, (((1,), (1,)), ((), ())),
                            preferred_element_type=jnp.float32)
    sq = jnp.sum(xt * xt, axis=1, keepdims=True)
    ones = jnp.ones((N, 1), jnp.float32)
    sq_row = jax.lax.dot_general(ones, sq, (((1,), (1,)), ((), ())),
                                 preferred_element_type=jnp.float32,
                                 precision=HI)
    score_ref[...] = (2.0 * g - sq) - sq_row
    m_ref[...] = jnp.full((N, o), NEG, jnp.float32)

    iota = jax.lax.broadcasted_iota(jnp.int32, (N, N), 1)

    def step(_, carry):
        sc = score_ref[...]
        mx = jnp.max(sc, axis=1, keepdims=True)
        cand = jnp.where(sc == mx, iota, N)
        amin = jnp.min(cand, axis=1, keepdims=True)
        hit = iota == amin
        score_ref[...] = jnp.where(hit, NEG, sc)
        onehot = hit.astype(jnp.float32)
        feat = jnp.dot(onehot, xt, preferred_element_type=jnp.float32,
                       precision=HI)
        db = (feat - xt).astype(jnp.bfloat16)
        hcat = jnp.concatenate([db, xb], axis=1)
        hj = jnp.dot(hcat, w_ref[...], preferred_element_type=jnp.float32)
        m_ref[...] = jnp.maximum(m_ref[...], s_ref[...] * hj + b_ref[...])
        return carry

    jax.lax.fori_loop(0, K, step, 0)
    out_ref[0] = _lrelu(m_ref[...])


def _stage(x, w, s, b):
    bsz, _, _ = x.shape
    o = w.shape[1]
    return pl.pallas_call(
        _stage_body,
        grid=(bsz,),
        in_specs=[
            pl.BlockSpec((1, N, x.shape[2]), lambda i: (i, 0, 0)),
            pl.BlockSpec(w.shape, lambda i: (0, 0)),
            pl.BlockSpec(s.shape, lambda i: (0, 0)),
            pl.BlockSpec(b.shape, lambda i: (0, 0)),
        ],
        out_specs=pl.BlockSpec((1, N, o), lambda i: (i, 0, 0)),
        out_shape=jax.ShapeDtypeStruct((bsz, N, o), jnp.float32),
        scratch_shapes=[
            pltpu.VMEM((N, N), jnp.float32),
            pltpu.VMEM((N, o), jnp.float32),
        ],
    )(x, w, s, b)


def _pool_body(x1_ref, x2_ref, x3_ref, x4_ref, w5_ref, s5_ref, b5_ref,
               out_ref):
    hc = jnp.concatenate(
        [x1_ref[0], x2_ref[0], x3_ref[0], x4_ref[0]], axis=1)
    h = jnp.dot(hc.astype(jnp.bfloat16), w5_ref[...],
                preferred_element_type=jnp.float32)
    h = _lrelu(s5_ref[...] * h + b5_ref[...])
    p1 = jnp.max(h, axis=0, keepdims=True)
    p2 = jnp.sum(h, axis=0, keepdims=True) * jnp.float32(1.0 / N)
    out_ref[0] = jnp.concatenate([p1, p2], axis=1)


def _pool(x1, x2, x3, x4, w5, s5, b5):
    bsz = x1.shape[0]
    specs = [pl.BlockSpec((1, N, v.shape[2]), lambda i: (i, 0, 0))
             for v in (x1, x2, x3, x4)]
    specs += [pl.BlockSpec(w5.shape, lambda i: (0, 0)),
              pl.BlockSpec(s5.shape, lambda i: (0, 0)),
              pl.BlockSpec(b5.shape, lambda i: (0, 0))]
    return pl.pallas_call(
        _pool_body,
        grid=(bsz,),
        in_specs=specs,
        out_specs=pl.BlockSpec((1, 1, 2048), lambda i: (i, 0, 0)),
        out_shape=jax.ShapeDtypeStruct((bsz, 1, 2048), jnp.float32),
    )(x1, x2, x3, x4, w5, s5, b5).reshape(bsz, 2048)


def _fc_body(z_ref, l1_ref, s6_ref, b6_ref, l2_ref, bl2_ref, s7_ref, b7_ref,
             l3_ref, bl3_ref, out_ref):
    z = z_ref[...]
    t = jnp.dot(z.astype(jnp.bfloat16), l1_ref[...],
                preferred_element_type=jnp.float32)
    t = _lrelu(s6_ref[...] * t + b6_ref[...])
    u = jnp.dot(t.astype(jnp.bfloat16), l2_ref[...],
                preferred_element_type=jnp.float32) + bl2_ref[...]
    u = _lrelu(s7_ref[...] * u + b7_ref[...])
    out_ref[...] = (jnp.dot(u.astype(jnp.bfloat16), l3_ref[...],
                            preferred_element_type=jnp.float32)
                    + bl3_ref[...])


def _fc(z, l1, s6, b6, l2, bl2, s7, b7, l3, bl3):
    return pl.pallas_call(
        _fc_body,
        out_shape=jax.ShapeDtypeStruct((z.shape[0], l3.shape[1]), jnp.float32),
    )(z, l1, s6, b6, l2, bl2, s7, b7, l3, bl3)


def _scale(g):
    return (g / jnp.sqrt(jnp.float32(1.0 + 1e-5)))[None, :]


@jax.jit
def kernel(x, W1, g1, b1, W2, g2, b2, W3, g3, b3, W4, g4, b4, W5, g5, b5,
           L1, g6, b6, L2, L2b, g7, b7, L3, L3b):
    xt = jnp.swapaxes(x, 1, 2)

    x1 = _stage(xt, W1.T.astype(jnp.bfloat16), _scale(g1), b1[None, :])
    x2 = _stage(x1, W2.T.astype(jnp.bfloat16), _scale(g2), b2[None, :])
    x3 = _stage(x2, W3.T.astype(jnp.bfloat16), _scale(g3), b3[None, :])
    x4 = _stage(x3, W4.T.astype(jnp.bfloat16), _scale(g4), b4[None, :])

    z = _pool(x1, x2, x3, x4, W5.T.astype(jnp.bfloat16), _scale(g5),
              b5[None, :])

    return _fc(z, L1.T.astype(jnp.bfloat16), _scale(g6), b6[None, :],
               L2.T.astype(jnp.bfloat16), L2b[None, :], _scale(g7),
               b7[None, :], L3.T.astype(jnp.bfloat16), L3b[None, :])


# speedup vs baseline: 8.3973x; 1.6443x over previous
import jax
import jax.numpy as jnp
from jax.experimental import pallas as pl
from jax.experimental.pallas import tpu as pltpu

N = 1024
K = 20
NEG = -3.0e38
HI = jax.lax.Precision.HIGHEST


def _lrelu(x):
    return jnp.where(x >= 0, x, 0.2 * x)


def _stage_body(x_ref, w_ref, s_ref, b_ref, out_ref, score_ref, m_ref):
    xt = x_ref[0]
    o = w_ref.shape[1]
    xb = xt.astype(jnp.bfloat16)

    g = jax.lax.dot_general(xb, xb, (((1,), (1,)), ((), ())),
                            preferred_element_type=jnp.float32)
    sq = jnp.sum(xt * xt, axis=1, keepdims=True)
    ones = jnp.ones((N, 1), jnp.float32)
    sq_row = jax.lax.dot_general(ones, sq, (((1,), (1,)), ((), ())),
                                 preferred_element_type=jnp.float32,
                                 precision=HI)
    score_ref[...] = (2.0 * g - sq) - sq_row
    m_ref[...] = jnp.full((N, o), NEG, jnp.float32)

    x_hi = xb
    r1 = xt - x_hi.astype(jnp.float32)
    x_mid = r1.astype(jnp.bfloat16)
    r2 = r1 - x_mid.astype(jnp.float32)
    x_lo = r2.astype(jnp.bfloat16)

    iota = jax.lax.broadcasted_iota(jnp.int32, (N, N), 1)

    def step(_, carry):
        sc = score_ref[...]
        mx = jnp.max(sc, axis=1, keepdims=True)
        cand = jnp.where(sc == mx, iota, N)
        amin = jnp.min(cand, axis=1, keepdims=True)
        hit = iota == amin
        score_ref[...] = jnp.where(hit, NEG, sc)
        ohb = hit.astype(jnp.bfloat16)
        feat = (jnp.dot(ohb, x_hi, preferred_element_type=jnp.float32)
                + jnp.dot(ohb, x_mid, preferred_element_type=jnp.float32)
                + jnp.dot(ohb, x_lo, preferred_element_type=jnp.float32))
        db = (feat - xt).astype(jnp.bfloat16)
        hcat = jnp.concatenate([db, xb], axis=1)
        hj = jnp.dot(hcat, w_ref[...], preferred_element_type=jnp.float32)
        m_ref[...] = jnp.maximum(m_ref[...], s_ref[...] * hj + b_ref[...])
        return carry

    jax.lax.fori_loop(0, K, step, 0)
    out_ref[0] = _lrelu(m_ref[...])


def _stage(x, w, s, b):
    bsz, _, _ = x.shape
    o = w.shape[1]
    return pl.pallas_call(
        _stage_body,
        grid=(bsz,),
        in_specs=[
            pl.BlockSpec((1, N, x.shape[2]), lambda i: (i, 0, 0)),
            pl.BlockSpec(w.shape, lambda i: (0, 0)),
            pl.BlockSpec(s.shape, lambda i: (0, 0)),
            pl.BlockSpec(b.shape, lambda i: (0, 0)),
        ],
        out_specs=pl.BlockSpec((1, N, o), lambda i: (i, 0, 0)),
        out_shape=jax.ShapeDtypeStruct((bsz, N, o), jnp.float32),
        scratch_shapes=[
            pltpu.VMEM((N, N), jnp.float32),
            pltpu.VMEM((N, o), jnp.float32),
        ],
    )(x, w, s, b)


def _pool_body(x1_ref, x2_ref, x3_ref, x4_ref, w5_ref, s5_ref, b5_ref,
               out_ref):
    hc = jnp.concatenate(
        [x1_ref[0], x2_ref[0], x3_ref[0], x4_ref[0]], axis=1)
    h = jnp.dot(hc.astype(jnp.bfloat16), w5_ref[...],
                preferred_element_type=jnp.float32)
    h = _lrelu(s5_ref[...] * h + b5_ref[...])
    p1 = jnp.max(h, axis=0, keepdims=True)
    p2 = jnp.sum(h, axis=0, keepdims=True) * jnp.float32(1.0 / N)
    out_ref[0] = jnp.concatenate([p1, p2], axis=1)


def _pool(x1, x2, x3, x4, w5, s5, b5):
    bsz = x1.shape[0]
    specs = [pl.BlockSpec((1, N, v.shape[2]), lambda i: (i, 0, 0))
             for v in (x1, x2, x3, x4)]
    specs += [pl.BlockSpec(w5.shape, lambda i: (0, 0)),
              pl.BlockSpec(s5.shape, lambda i: (0, 0)),
              pl.BlockSpec(b5.shape, lambda i: (0, 0))]
    return pl.pallas_call(
        _pool_body,
        grid=(bsz,),
        in_specs=specs,
        out_specs=pl.BlockSpec((1, 1, 2048), lambda i: (i, 0, 0)),
        out_shape=jax.ShapeDtypeStruct((bsz, 1, 2048), jnp.float32),
    )(x1, x2, x3, x4, w5, s5, b5).reshape(bsz, 2048)


def _fc_body(z_ref, l1_ref, s6_ref, b6_ref, l2_ref, bl2_ref, s7_ref, b7_ref,
             l3_ref, bl3_ref, out_ref):
    z = z_ref[...]
    t = jnp.dot(z.astype(jnp.bfloat16), l1_ref[...],
                preferred_element_type=jnp.float32)
    t = _lrelu(s6_ref[...] * t + b6_ref[...])
    u = jnp.dot(t.astype(jnp.bfloat16), l2_ref[...],
                preferred_element_type=jnp.float32) + bl2_ref[...]
    u = _lrelu(s7_ref[...] * u + b7_ref[...])
    out_ref[...] = (jnp.dot(u.astype(jnp.bfloat16), l3_ref[...],
                            preferred_element_type=jnp.float32)
                    + bl3_ref[...])


def _fc(z, l1, s6, b6, l2, bl2, s7, b7, l3, bl3):
    return pl.pallas_call(
        _fc_body,
        out_shape=jax.ShapeDtypeStruct((z.shape[0], l3.shape[1]), jnp.float32),
    )(z, l1, s6, b6, l2, bl2, s7, b7, l3, bl3)


def _scale(g):
    return (g / jnp.sqrt(jnp.float32(1.0 + 1e-5)))[None, :]


@jax.jit
def kernel(x, W1, g1, b1, W2, g2, b2, W3, g3, b3, W4, g4, b4, W5, g5, b5,
           L1, g6, b6, L2, L2b, g7, b7, L3, L3b):
    xt = jnp.swapaxes(x, 1, 2)

    x1 = _stage(xt, W1.T.astype(jnp.bfloat16), _scale(g1), b1[None, :])
    x2 = _stage(x1, W2.T.astype(jnp.bfloat16), _scale(g2), b2[None, :])
    x3 = _stage(x2, W3.T.astype(jnp.bfloat16), _scale(g3), b3[None, :])
    x4 = _stage(x3, W4.T.astype(jnp.bfloat16), _scale(g4), b4[None, :])

    z = _pool(x1, x2, x3, x4, W5.T.astype(jnp.bfloat16), _scale(g5),
              b5[None, :])

    return _fc(z, L1.T.astype(jnp.bfloat16), _scale(g6), b6[None, :],
               L2.T.astype(jnp.bfloat16), L2b[None, :], _scale(g7),
               b7[None, :], L3.T.astype(jnp.bfloat16), L3b[None, :])


# speedup vs baseline: 10.0961x; 1.2023x over previous
import functools

import jax
import jax.numpy as jnp
from jax import lax
from jax.experimental import pallas as pl
from jax.experimental.pallas import tpu as pltpu
from jax.experimental.pallas import tpu_sc as plsc

N = 1024
K = 20
NEG = -3.0e38
HI = jax.lax.Precision.HIGHEST
GCHUNK = 128


def _lrelu(x):
    return jnp.where(x >= 0, x, 0.2 * x)


def _topk_body(x_ref, idx_ref, score_ref):
    xt = x_ref[0]
    xb = xt.astype(jnp.bfloat16)

    g = jax.lax.dot_general(xb, xb, (((1,), (1,)), ((), ())),
                            preferred_element_type=jnp.float32)
    sq = jnp.sum(xt * xt, axis=1, keepdims=True)
    ones = jnp.ones((N, 1), jnp.float32)
    sq_row = jax.lax.dot_general(ones, sq, (((1,), (1,)), ((), ())),
                                 preferred_element_type=jnp.float32,
                                 precision=HI)
    score_ref[...] = (2.0 * g - sq) - sq_row

    iota = jax.lax.broadcasted_iota(jnp.int32, (N, N), 1)
    cols = []
    for _ in range(K):
        sc = score_ref[...]
        mx = jnp.max(sc, axis=1, keepdims=True)
        cand = jnp.where(sc == mx, iota, N)
        amin = jnp.min(cand, axis=1, keepdims=True)
        score_ref[...] = jnp.where(iota == amin, NEG, sc)
        cols.append(amin)
    amat = jnp.concatenate(cols, axis=1)
    amat = amat + pl.program_id(0) * N
    idx_ref[0] = jnp.transpose(amat)


def _topk(x):
    bsz = x.shape[0]
    return pl.pallas_call(
        _topk_body,
        grid=(bsz,),
        in_specs=[pl.BlockSpec((1, N, x.shape[2]), lambda i: (i, 0, 0))],
        out_specs=pl.BlockSpec((1, K, N), lambda i: (i, 0, 0)),
        out_shape=jax.ShapeDtypeStruct((bsz, K, N), jnp.int32),
        scratch_shapes=[pltpu.VMEM((N, N), jnp.float32)],
    )(x)


def _sc_gather(x_flat, idx_flat):
    t_total, _ = idx_flat.shape
    c = x_flat.shape[1]
    info = plsc.get_sparse_core_info()
    nc, ns = info.num_cores, info.num_subcores
    nw = nc * ns
    n_rounds = (t_total + nw - 1) // nw
    mesh = plsc.VectorSubcoreMesh(core_axis_name="c", subcore_axis_name="s")

    @functools.partial(
        pl.kernel, mesh=mesh,
        compiler_params=pltpu.CompilerParams(use_tc_tiling_on_sc=False),
        out_type=jax.ShapeDtypeStruct((t_total, N, c), jnp.float32),
        scratch_types=[
            pltpu.VMEM((GCHUNK,), jnp.int32),
            pltpu.VMEM((GCHUNK, c), jnp.float32),
            pltpu.SemaphoreType.DMA,
        ],
    )
    def gk(x_hbm, idx_hbm, out_hbm, idx_v, rows_v, sem):
        wid = lax.axis_index("s") * nc + lax.axis_index("c")
        for r in range(n_rounds):
            t = wid + r * nw

            @pl.when(t < t_total)
            def _do():
                for ch in range(N // GCHUNK):
                    pltpu.sync_copy(
                        idx_hbm.at[t, pl.ds(ch * GCHUNK, GCHUNK)], idx_v)
                    pltpu.async_copy(x_hbm.at[idx_v], rows_v, sem).wait()
                    pltpu.sync_copy(
                        rows_v, out_hbm.at[t, pl.ds(ch * GCHUNK, GCHUNK)])

    return gk(x_flat, idx_flat)


def _edge_body(x_ref, feat_ref, w_ref, s_ref, b_ref, out_ref, m_ref):
    xt = x_ref[0]
    o = w_ref.shape[1]
    cdim = xt.shape[1]
    xb = xt.astype(jnp.bfloat16)
    m_ref[...] = jnp.full((N, o), NEG, jnp.float32)
    for j in range(K):
        d = (feat_ref[j][:, :cdim] - xt).astype(jnp.bfloat16)
        hcat = jnp.concatenate([d, xb], axis=1)
        hj = jnp.dot(hcat, w_ref[...], preferred_element_type=jnp.float32)
        m_ref[...] = jnp.maximum(m_ref[...], s_ref[...] * hj + b_ref[...])
    out_ref[0] = _lrelu(m_ref[...])


def _edge(x, feat, w, s, b):
    bsz, _, cdim = x.shape
    o = w.shape[1]
    return pl.pallas_call(
        _edge_body,
        grid=(bsz,),
        in_specs=[
            pl.BlockSpec((1, N, cdim), lambda i: (i, 0, 0)),
            pl.BlockSpec((K, N, feat.shape[2]), lambda i: (i, 0, 0)),
            pl.BlockSpec(w.shape, lambda i: (0, 0)),
            pl.BlockSpec(s.shape, lambda i: (0, 0)),
            pl.BlockSpec(b.shape, lambda i: (0, 0)),
        ],
        out_specs=pl.BlockSpec((1, N, o), lambda i: (i, 0, 0)),
        out_shape=jax.ShapeDtypeStruct((bsz, N, o), jnp.float32),
        scratch_shapes=[pltpu.VMEM((N, o), jnp.float32)],
    )(x, feat, w, s, b)


def _stage(x, w, s, b):
    bsz, _, cdim = x.shape
    idx = _topk(x)
    xf = x.reshape(bsz * N, cdim)
    if cdim % 8:
        xf = jnp.pad(xf, ((0, 0), (0, 8 - cdim % 8)))
    feat = _sc_gather(xf, idx.reshape(bsz * K, N))
    return _edge(x, feat, w, s, b)


def _pool_body(x1_ref, x2_ref, x3_ref, x4_ref, w5_ref, s5_ref, b5_ref,
               out_ref):
    hc = jnp.concatenate(
        [x1_ref[0], x2_ref[0], x3_ref[0], x4_ref[0]], axis=1)
    h = jnp.dot(hc.astype(jnp.bfloat16), w5_ref[...],
                preferred_element_type=jnp.float32)
    h = _lrelu(s5_ref[...] * h + b5_ref[...])
    p1 = jnp.max(h, axis=0, keepdims=True)
    p2 = jnp.sum(h, axis=0, keepdims=True) * jnp.float32(1.0 / N)
    out_ref[0] = jnp.concatenate([p1, p2], axis=1)


def _pool(x1, x2, x3, x4, w5, s5, b5):
    bsz = x1.shape[0]
    specs = [pl.BlockSpec((1, N, v.shape[2]), lambda i: (i, 0, 0))
             for v in (x1, x2, x3, x4)]
    specs += [pl.BlockSpec(w5.shape, lambda i: (0, 0)),
              pl.BlockSpec(s5.shape, lambda i: (0, 0)),
              pl.BlockSpec(b5.shape, lambda i: (0, 0))]
    return pl.pallas_call(
        _pool_body,
        grid=(bsz,),
        in_specs=specs,
        out_specs=pl.BlockSpec((1, 1, 2048), lambda i: (i, 0, 0)),
        out_shape=jax.ShapeDtypeStruct((bsz, 1, 2048), jnp.float32),
    )(x1, x2, x3, x4, w5, s5, b5).reshape(bsz, 2048)


def _fc_body(z_ref, l1_ref, s6_ref, b6_ref, l2_ref, bl2_ref, s7_ref, b7_ref,
             l3_ref, bl3_ref, out_ref):
    z = z_ref[...]
    t = jnp.dot(z.astype(jnp.bfloat16), l1_ref[...],
                preferred_element_type=jnp.float32)
    t = _lrelu(s6_ref[...] * t + b6_ref[...])
    u = jnp.dot(t.astype(jnp.bfloat16), l2_ref[...],
                preferred_element_type=jnp.float32) + bl2_ref[...]
    u = _lrelu(s7_ref[...] * u + b7_ref[...])
    out_ref[...] = (jnp.dot(u.astype(jnp.bfloat16), l3_ref[...],
                            preferred_element_type=jnp.float32)
                    + bl3_ref[...])


def _fc(z, l1, s6, b6, l2, bl2, s7, b7, l3, bl3):
    return pl.pallas_call(
        _fc_body,
        out_shape=jax.ShapeDtypeStruct((z.shape[0], l3.shape[1]), jnp.float32),
    )(z, l1, s6, b6, l2, bl2, s7, b7, l3, bl3)


def _scale(g):
    return (g / jnp.sqrt(jnp.float32(1.0 + 1e-5)))[None, :]


@jax.jit
def kernel(x, W1, g1, b1, W2, g2, b2, W3, g3, b3, W4, g4, b4, W5, g5, b5,
           L1, g6, b6, L2, L2b, g7, b7, L3, L3b):
    xt = jnp.swapaxes(x, 1, 2)

    x1 = _stage(xt, W1.T.astype(jnp.bfloat16), _scale(g1), b1[None, :])
    x2 = _stage(x1, W2.T.astype(jnp.bfloat16), _scale(g2), b2[None, :])
    x3 = _stage(x2, W3.T.astype(jnp.bfloat16), _scale(g3), b3[None, :])
    x4 = _stage(x3, W4.T.astype(jnp.bfloat16), _scale(g4), b4[None, :])

    z = _pool(x1, x2, x3, x4, W5.T.astype(jnp.bfloat16), _scale(g5),
              b5[None, :])

    return _fc(z, L1.T.astype(jnp.bfloat16), _scale(g6), b6[None, :],
               L2.T.astype(jnp.bfloat16), L2b[None, :], _scale(g7),
               b7[None, :], L3.T.astype(jnp.bfloat16), L3b[None, :])


# speedup vs baseline: 11.2656x; 1.1158x over previous
import functools

import jax
import jax.numpy as jnp
from jax import lax
from jax.experimental import pallas as pl
from jax.experimental.pallas import tpu as pltpu
from jax.experimental.pallas import tpu_sc as plsc

N = 1024
K = 20
NEG = -3.0e38
HI = jax.lax.Precision.HIGHEST
GCHUNK = 128


def _lrelu(x):
    return jnp.where(x >= 0, x, 0.2 * x)


def _topk_body(x_ref, idx_ref, score_ref):
    xt = x_ref[0]
    xb = xt.astype(jnp.bfloat16)

    g = jax.lax.dot_general(xb, xb, (((1,), (1,)), ((), ())),
                            preferred_element_type=jnp.float32)
    sq = jnp.sum(xt * xt, axis=1, keepdims=True)
    ones = jnp.ones((N, 1), jnp.float32)
    sq_row = jax.lax.dot_general(ones, sq, (((1,), (1,)), ((), ())),
                                 preferred_element_type=jnp.float32,
                                 precision=HI)
    score_ref[...] = (2.0 * g - sq) - sq_row

    iota = jax.lax.broadcasted_iota(jnp.int32, (N, N), 1)
    cols = []
    for _ in range(K):
        sc = score_ref[...]
        mx = jnp.max(sc, axis=1, keepdims=True)
        cand = jnp.where(sc == mx, iota, N)
        amin = jnp.min(cand, axis=1, keepdims=True)
        score_ref[...] = jnp.where(iota == amin, NEG, sc)
        cols.append(amin)
    amat = jnp.concatenate(cols, axis=1)
    amat = amat + pl.program_id(0) * N
    idx_ref[0] = jnp.transpose(amat)


def _topk(x):
    bsz = x.shape[0]
    return pl.pallas_call(
        _topk_body,
        grid=(bsz,),
        in_specs=[pl.BlockSpec((1, N, x.shape[2]), lambda i: (i, 0, 0))],
        out_specs=pl.BlockSpec((1, K, N), lambda i: (i, 0, 0)),
        out_shape=jax.ShapeDtypeStruct((bsz, K, N), jnp.int32),
        scratch_shapes=[pltpu.VMEM((N, N), jnp.float32)],
    )(x)


def _sc_gather(x_flat, idx_flat):
    t_total, _ = idx_flat.shape
    c = x_flat.shape[1]
    info = plsc.get_sparse_core_info()
    nc, ns = info.num_cores, info.num_subcores
    nw = nc * ns
    half = N // 4
    n_rounds = (4 * t_total) // nw
    mesh = plsc.VectorSubcoreMesh(core_axis_name="c", subcore_axis_name="s")

    @functools.partial(
        pl.kernel, mesh=mesh,
        compiler_params=pltpu.CompilerParams(use_tc_tiling_on_sc=False),
        out_type=jax.ShapeDtypeStruct((t_total, N, c), jnp.float32),
        scratch_types=[
            pltpu.VMEM((half,), jnp.int32),
            pltpu.VMEM((half, c), jnp.float32),
            pltpu.VMEM((half, c), jnp.float32),
            pltpu.SemaphoreType.DMA,
            pltpu.SemaphoreType.DMA,
            pltpu.SemaphoreType.DMA,
        ],
    )
    def gk(x_hbm, idx_hbm, out_hbm, idx_v, rows0, rows1, semg, semw0, semw1):
        wid = lax.axis_index("s") * nc + lax.axis_index("c")
        rows = (rows0, rows1)
        semw = (semw0, semw1)
        pending = [None, None]
        for r in range(n_rounds):
            h = wid * n_rounds + r
            t = h // 4
            off = (h % 4) * half
            buf = rows[r % 2]
            pltpu.sync_copy(idx_hbm.at[t, pl.ds(off, half)], idx_v)
            if pending[r % 2] is not None:
                pending[r % 2].wait()
            gathers = []
            for ch in range(half // GCHUNK):
                gathers.append(pltpu.async_copy(
                    x_hbm.at[idx_v.at[pl.ds(ch * GCHUNK, GCHUNK)]],
                    buf.at[pl.ds(ch * GCHUNK, GCHUNK)], semg))
            for gcp in gathers:
                gcp.wait()
            pending[r % 2] = pltpu.async_copy(
                buf, out_hbm.at[t, pl.ds(off, half)], semw[r % 2])
        for p in pending:
            if p is not None:
                p.wait()

    return gk(x_flat, idx_flat)


def _edge_body(x_ref, feat_ref, w_ref, s_ref, b_ref, out_ref, m_ref):
    xt = x_ref[0]
    o = w_ref.shape[1]
    cdim = xt.shape[1]
    xb = xt.astype(jnp.bfloat16)
    m_ref[...] = jnp.full((N, o), NEG, jnp.float32)
    for j in range(K):
        d = (feat_ref[j][:, :cdim] - xt).astype(jnp.bfloat16)
        hcat = jnp.concatenate([d, xb], axis=1)
        hj = jnp.dot(hcat, w_ref[...], preferred_element_type=jnp.float32)
        m_ref[...] = jnp.maximum(m_ref[...], s_ref[...] * hj + b_ref[...])
    out_ref[0] = _lrelu(m_ref[...])


def _edge(x, feat, w, s, b):
    bsz, _, cdim = x.shape
    o = w.shape[1]
    return pl.pallas_call(
        _edge_body,
        grid=(bsz,),
        in_specs=[
            pl.BlockSpec((1, N, cdim), lambda i: (i, 0, 0)),
            pl.BlockSpec((K, N, feat.shape[2]), lambda i: (i, 0, 0)),
            pl.BlockSpec(w.shape, lambda i: (0, 0)),
            pl.BlockSpec(s.shape, lambda i: (0, 0)),
            pl.BlockSpec(b.shape, lambda i: (0, 0)),
        ],
        out_specs=pl.BlockSpec((1, N, o), lambda i: (i, 0, 0)),
        out_shape=jax.ShapeDtypeStruct((bsz, N, o), jnp.float32),
        scratch_shapes=[pltpu.VMEM((N, o), jnp.float32)],
    )(x, feat, w, s, b)


def _stage(x, w, s, b):
    bsz, _, cdim = x.shape
    idx = _topk(x)
    xf = x.reshape(bsz * N, cdim)
    if cdim % 8:
        xf = jnp.pad(xf, ((0, 0), (0, 8 - cdim % 8)))
    feat = _sc_gather(xf, idx.reshape(bsz * K, N))
    return _edge(x, feat, w, s, b)


def _pool_body(x1_ref, x2_ref, x3_ref, x4_ref, w5_ref, s5_ref, b5_ref,
               out_ref):
    hc = jnp.concatenate(
        [x1_ref[0], x2_ref[0], x3_ref[0], x4_ref[0]], axis=1)
    h = jnp.dot(hc.astype(jnp.bfloat16), w5_ref[...],
                preferred_element_type=jnp.float32)
    h = _lrelu(s5_ref[...] * h + b5_ref[...])
    p1 = jnp.max(h, axis=0, keepdims=True)
    p2 = jnp.sum(h, axis=0, keepdims=True) * jnp.float32(1.0 / N)
    out_ref[0] = jnp.concatenate([p1, p2], axis=1)


def _pool(x1, x2, x3, x4, w5, s5, b5):
    bsz = x1.shape[0]
    specs = [pl.BlockSpec((1, N, v.shape[2]), lambda i: (i, 0, 0))
             for v in (x1, x2, x3, x4)]
    specs += [pl.BlockSpec(w5.shape, lambda i: (0, 0)),
              pl.BlockSpec(s5.shape, lambda i: (0, 0)),
              pl.BlockSpec(b5.shape, lambda i: (0, 0))]
    return pl.pallas_call(
        _pool_body,
        grid=(bsz,),
        in_specs=specs,
        out_specs=pl.BlockSpec((1, 1, 2048), lambda i: (i, 0, 0)),
        out_shape=jax.ShapeDtypeStruct((bsz, 1, 2048), jnp.float32),
    )(x1, x2, x3, x4, w5, s5, b5).reshape(bsz, 2048)


def _fc_body(z_ref, l1_ref, s6_ref, b6_ref, l2_ref, bl2_ref, s7_ref, b7_ref,
             l3_ref, bl3_ref, out_ref):
    z = z_ref[...]
    t = jnp.dot(z.astype(jnp.bfloat16), l1_ref[...],
                preferred_element_type=jnp.float32)
    t = _lrelu(s6_ref[...] * t + b6_ref[...])
    u = jnp.dot(t.astype(jnp.bfloat16), l2_ref[...],
                preferred_element_type=jnp.float32) + bl2_ref[...]
    u = _lrelu(s7_ref[...] * u + b7_ref[...])
    out_ref[...] = (jnp.dot(u.astype(jnp.bfloat16), l3_ref[...],
                            preferred_element_type=jnp.float32)
                    + bl3_ref[...])


def _fc(z, l1, s6, b6, l2, bl2, s7, b7, l3, bl3):
    return pl.pallas_call(
        _fc_body,
        out_shape=jax.ShapeDtypeStruct((z.shape[0], l3.shape[1]), jnp.float32),
    )(z, l1, s6, b6, l2, bl2, s7, b7, l3, bl3)


def _scale(g):
    return (g / jnp.sqrt(jnp.float32(1.0 + 1e-5)))[None, :]


@jax.jit
def kernel(x, W1, g1, b1, W2, g2, b2, W3, g3, b3, W4, g4, b4, W5, g5, b5,
           L1, g6, b6, L2, L2b, g7, b7, L3, L3b):
    xt = jnp.swapaxes(x, 1, 2)

    x1 = _stage(xt, W1.T.astype(jnp.bfloat16), _scale(g1), b1[None, :])
    x2 = _stage(x1, W2.T.astype(jnp.bfloat16), _scale(g2), b2[None, :])
    x3 = _stage(x2, W3.T.astype(jnp.bfloat16), _scale(g3), b3[None, :])
    x4 = _stage(x3, W4.T.astype(jnp.bfloat16), _scale(g4), b4[None, :])

    z = _pool(x1, x2, x3, x4, W5.T.astype(jnp.bfloat16), _scale(g5),
              b5[None, :])

    return _fc(z, L1.T.astype(jnp.bfloat16), _scale(g6), b6[None, :],
               L2.T.astype(jnp.bfloat16), L2b[None, :], _scale(g7),
               b7[None, :], L3.T.astype(jnp.bfloat16), L3b[None, :])
